# baseline (speedup 1.0000x reference)
import functools

import jax
import jax.numpy as jnp
from jax import lax
from jax.experimental import pallas as pl
from jax.experimental.pallas import tpu as pltpu
from jax.experimental.pallas import tpu_sc as plsc

D_MODEL = 64
SCALE = 8.0
GRP = 128
K = 4
NC = 2
NS = 16
NW = NC * NS


def _emb_body(x_hbm, table_hbm, out_hbm, idx_v, rows_v, sem):
    c = lax.axis_index("c")
    s = lax.axis_index("s")
    wid = s * NC + c
    groups_per_worker = x_hbm.shape[0] // NW
    nchunks = groups_per_worker // K

    def chunk_body(t, carry):
        rb = wid * groups_per_worker + t * K
        pltpu.sync_copy(x_hbm.at[pl.ds(rb, K)], idx_v)
        cps = [
            pltpu.async_copy(table_hbm.at[idx_v.at[j]], rows_v.at[j], sem)
            for j in range(K)
        ]
        for cp in cps:
            cp.wait()

        def scale_row(r, _):
            for j in range(K):
                for cix in range(D_MODEL // 16):
                    sl = pl.ds(cix * 16, 16)
                    rows_v[j, r, sl] = rows_v[j, r, sl] * SCALE
            return 0

        lax.fori_loop(0, GRP, scale_row, 0, unroll=2)
        pltpu.sync_copy(rows_v, out_hbm.at[pl.ds(rb, K)])
        return carry

    lax.fori_loop(0, nchunks, chunk_body, 0)


@jax.jit
def kernel(x, table):
    orig_shape = x.shape
    b = x.size
    assert b % (NW * GRP * K) == 0
    ngroups = b // GRP
    xi = x.reshape(ngroups, GRP).astype(jnp.int32)

    run = pl.kernel(
        _emb_body,
        out_type=jax.ShapeDtypeStruct((ngroups, GRP, D_MODEL), jnp.float32),
        mesh=plsc.VectorSubcoreMesh(core_axis_name="c", subcore_axis_name="s"),
        scratch_types=[
            pltpu.VMEM((K, GRP), jnp.int32),
            pltpu.VMEM((K, GRP, D_MODEL), jnp.float32),
            pltpu.SemaphoreType.DMA,
        ],
        compiler_params=pltpu.CompilerParams(use_tc_tiling_on_sc=False),
    )
    out = run(xi, table)
    return out.reshape(*orig_shape, D_MODEL)


# speedup vs baseline: 1.0927x; 1.0927x over previous
import jax
import jax.numpy as jnp
from jax import lax
from jax.experimental import pallas as pl
from jax.experimental.pallas import tpu as pltpu
from jax.experimental.pallas import tpu_sc as plsc

D_MODEL = 64
SCALE = 8.0
GRP = 128
K = 2
NBUF = 4
NC = 2
NS = 16
NW = NC * NS


def _emb_body(x_hbm, table_hbm, out_hbm, idx_v, bufs, gsem, osem):
    c = lax.axis_index("c")
    s = lax.axis_index("s")
    wid = s * NC + c
    gpw = x_hbm.shape[0] // NW
    nchunks = gpw // K
    grp_base = wid * gpw

    pltpu.sync_copy(x_hbm.at[pl.ds(grp_base, gpw)], idx_v)

    def fire(ch, b):
        for j in range(K):
            pltpu.async_copy(
                table_hbm.at[idx_v.at[ch * K + j]],
                bufs.at[b, j],
                gsem.at[b],
            )

    def wait_gather(b):
        pltpu.make_async_copy(
            out_hbm.at[pl.ds(0, K)], bufs.at[b], gsem.at[b]
        ).wait()

    def fire_out(ch, b):
        pltpu.async_copy(
            bufs.at[b], out_hbm.at[pl.ds(grp_base + ch * K, K)],
            osem.at[b],
        )

    def wait_out(b):
        pltpu.make_async_copy(
            bufs.at[b], out_hbm.at[pl.ds(0, K)], osem.at[b]
        ).wait()

    def scale(b):
        def scale_row(r, _):
            for j in range(K):
                for cix in range(D_MODEL // 16):
                    sl = pl.ds(cix * 16, 16)
                    bufs[b, j, r, sl] = bufs[b, j, r, sl] * SCALE
            return 0

        lax.fori_loop(0, GRP, scale_row, 0, unroll=4)

    for b in range(NBUF - 1):
        fire(b, b)

    def outer(i, carry):
        for b in range(NBUF):
            ch = i * NBUF + b
            nb = (b + NBUF - 1) % NBUF
            nch = ch + NBUF - 1

            @pl.when(jnp.logical_and(nch < nchunks, nch >= NBUF))
            def _():
                wait_out(nb)
                fire(nch, nb)

            @pl.when(jnp.logical_and(nch < nchunks, nch < NBUF))
            def _():
                fire(nch, nb)

            wait_gather(b)
            scale(b)
            fire_out(ch, b)
        return carry

    lax.fori_loop(0, nchunks // NBUF, outer, 0)

    for b in range(NBUF):
        wait_out(b)


@jax.jit
def kernel(x, table):
    orig_shape = x.shape
    b = x.size
    assert b % (NW * GRP * K * NBUF) == 0
    ngroups = b // GRP
    xi = x.reshape(ngroups, GRP).astype(jnp.int32)

    run = pl.kernel(
        _emb_body,
        out_type=jax.ShapeDtypeStruct((ngroups, GRP, D_MODEL), jnp.float32),
        mesh=plsc.VectorSubcoreMesh(core_axis_name="c", subcore_axis_name="s"),
        scratch_types=[
            pltpu.VMEM((ngroups // NW, GRP), jnp.int32),
            pltpu.VMEM((NBUF, K, GRP, D_MODEL), jnp.float32),
            pltpu.SemaphoreType.DMA((NBUF,)),
            pltpu.SemaphoreType.DMA((NBUF,)),
        ],
        compiler_params=pltpu.CompilerParams(use_tc_tiling_on_sc=False),
    )
    out = run(xi, table)
    return out.reshape(*orig_shape, D_MODEL)
